# baseline (speedup 1.0000x reference)
import functools

import jax
import jax.numpy as jnp
from jax import lax
from jax.experimental import pallas as pl
from jax.experimental.pallas import tpu as pltpu
from jax.experimental.pallas import tpu_sc as plsc

_TM = 512
_TKC = 2048
_BETA = 0.25


def _vq_body(zb, wtb, zsqb, wsqb, idx_out, commit_out, codebook_out,
             best_d, best_t, best_i, acc, *, n_elem):
    i = pl.program_id(0)
    j = pl.program_id(1)
    ni = pl.num_programs(0)
    nj = pl.num_programs(1)

    dot = jnp.dot(zb[...].astype(jnp.bfloat16), wtb[...].astype(jnp.bfloat16),
                  preferred_element_type=jnp.float32)
    d2 = (zsqb[...] + wsqb[...]) - 2.0 * dot
    zdist = jnp.sqrt(jnp.maximum(d2, 0.0))

    bmin = jnp.min(zdist, axis=1, keepdims=True)
    ids = lax.broadcasted_iota(jnp.int32, (_TM, _TKC), 1) + j * _TKC
    big = jnp.int32(2**31 - 1)
    bidx = jnp.min(jnp.where(zdist == bmin, ids, big), axis=1, keepdims=True)
    bmin16 = bmin.astype(jnp.bfloat16).astype(jnp.float32)

    @pl.when(j == 0)
    def _():
        best_d[...] = bmin16
        best_t[...] = bmin
        best_i[...] = bidx

    @pl.when(j > 0)
    def _():
        cur_d = best_d[...]
        cur_i = best_i[...]
        keep = (cur_d < bmin) | ((cur_d == bmin) & (cur_i < bidx))
        best_i[...] = jnp.where(keep, cur_i, bidx)
        best_t[...] = jnp.where(keep, best_t[...], bmin)
        best_d[...] = jnp.where(keep, cur_d, bmin16)

    @pl.when(j == nj - 1)
    def _():
        idx_out[...] = best_i[...]
        bv = best_t[...]
        s = jnp.sum(bv * bv).reshape(1, 1)

        @pl.when(i == 0)
        def _():
            acc[...] = s

        @pl.when(i > 0)
        def _():
            acc[...] = acc[...] + s

        @pl.when(i == ni - 1)
        def _():
            mean_sq = acc[...] / n_elem
            commit_out[...] = _BETA * mean_sq
            codebook_out[...] = mean_sq


def _vq_tc(z_flat, wt, z_sq, w_sq):
    n, d = z_flat.shape
    k = wt.shape[1]
    grid = (n // _TM, k // _TKC)
    body = functools.partial(_vq_body, n_elem=float(n * d))
    return pl.pallas_call(
        body,
        grid=grid,
        in_specs=[
            pl.BlockSpec((_TM, d), lambda i, j: (i, 0)),
            pl.BlockSpec((d, _TKC), lambda i, j: (0, j)),
            pl.BlockSpec((_TM, 1), lambda i, j: (i, 0)),
            pl.BlockSpec((1, _TKC), lambda i, j: (0, j)),
        ],
        out_specs=[
            pl.BlockSpec((_TM, 1), lambda i, j: (i, 0)),
            pl.BlockSpec((1, 1), lambda i, j: (0, 0)),
            pl.BlockSpec((1, 1), lambda i, j: (0, 0)),
        ],
        out_shape=[
            jax.ShapeDtypeStruct((n, 1), jnp.int32),
            jax.ShapeDtypeStruct((1, 1), jnp.float32),
            jax.ShapeDtypeStruct((1, 1), jnp.float32),
        ],
        scratch_shapes=[
            pltpu.VMEM((_TM, 1), jnp.float32),
            pltpu.VMEM((_TM, 1), jnp.float32),
            pltpu.VMEM((_TM, 1), jnp.int32),
            pltpu.VMEM((1, 1), jnp.float32),
        ],
    )(z_flat, wt, z_sq, w_sq)


def _sc_gather(table, idx):
    v, d0 = table.shape
    d = 128
    table = jnp.pad(table, ((0, 0), (0, d - d0)))
    b = idx.shape[0]
    info = plsc.get_sparse_core_info()
    nc, ns = info.num_cores, info.num_subcores
    nw = nc * ns
    b_per_w = b // nw
    mesh = plsc.VectorSubcoreMesh(core_axis_name="c", subcore_axis_name="s")

    @functools.partial(
        pl.kernel,
        mesh=mesh,
        out_type=jax.ShapeDtypeStruct((b, d), jnp.float32),
        scratch_types=[
            pltpu.VMEM((b_per_w,), jnp.int32),
            pltpu.VMEM((b_per_w, d), jnp.float32),
            pltpu.SemaphoreType.DMA,
        ],
    )
    def k(table_hbm, idx_hbm, out_hbm, idx_v, rows_v, sem):
        wid = lax.axis_index("s") * nc + lax.axis_index("c")
        base = wid * b_per_w
        pltpu.sync_copy(idx_hbm.at[pl.ds(base, b_per_w)], idx_v)
        pltpu.async_copy(table_hbm.at[idx_v], rows_v, sem).wait()
        pltpu.sync_copy(rows_v, out_hbm.at[pl.ds(base, b_per_w)])

    return k(table, idx)[:, :d0]


def kernel(z, W):
    e_dim = z.shape[-1]
    z_flat = z.reshape(-1, e_dim)
    z_sq = jnp.sum(z_flat * z_flat, axis=-1, keepdims=True)
    w_sq = jnp.sum(W * W, axis=-1)[None, :]
    wt = W.T
    idx2, commit, codebook = _vq_tc(z_flat, wt, z_sq, w_sq)
    indices = idx2.reshape(-1)
    z_q = _sc_gather(W, indices)
    z_q_st = z_q.reshape(z.shape)
    return (z_q_st, commit.reshape(()), codebook.reshape(()), indices)


# speedup vs baseline: 1.0385x; 1.0385x over previous
import functools

import jax
import jax.numpy as jnp
from jax import lax
from jax.experimental import pallas as pl
from jax.experimental.pallas import tpu as pltpu
from jax.experimental.pallas import tpu_sc as plsc

_TM = 512
_TKC = 2048
_BETA = 0.25


def _vq_body(zb, wtb, zsqb, wsqb, idx_out, commit_out, codebook_out,
             best_d, best_t, best_i, acc, *, n_elem):
    i = pl.program_id(0)
    j = pl.program_id(1)
    ni = pl.num_programs(0)
    nj = pl.num_programs(1)

    dot = jnp.dot(zb[...].astype(jnp.bfloat16), wtb[...].astype(jnp.bfloat16),
                  preferred_element_type=jnp.float32)
    d2 = (zsqb[...] + wsqb[...]) + dot
    zdist = jnp.sqrt(jnp.maximum(d2, 0.0))

    bmin = jnp.min(zdist, axis=1, keepdims=True)
    ids = lax.broadcasted_iota(jnp.int32, (_TM, _TKC), 1)
    big = jnp.int32(2**31 - 1)
    bidx = jnp.min(jnp.where(zdist == bmin, ids, big), axis=1,
                   keepdims=True) + j * _TKC
    bmin16 = bmin.astype(jnp.bfloat16).astype(jnp.float32)

    @pl.when(j == 0)
    def _():
        best_d[...] = bmin16
        best_t[...] = bmin
        best_i[...] = bidx

    @pl.when(j > 0)
    def _():
        cur_d = best_d[...]
        cur_i = best_i[...]
        keep = (cur_d < bmin) | ((cur_d == bmin) & (cur_i < bidx))
        best_i[...] = jnp.where(keep, cur_i, bidx)
        best_t[...] = jnp.where(keep, best_t[...], bmin)
        best_d[...] = jnp.where(keep, cur_d, bmin16)

    @pl.when(j == nj - 1)
    def _():
        idx_out[...] = best_i[...]
        bv = best_t[...]
        s = jnp.sum(bv * bv).reshape(1, 1)

        @pl.when(i == 0)
        def _():
            acc[...] = s

        @pl.when(i > 0)
        def _():
            acc[...] = acc[...] + s

        @pl.when(i == ni - 1)
        def _():
            mean_sq = acc[...] / n_elem
            commit_out[...] = _BETA * mean_sq
            codebook_out[...] = mean_sq


def _vq_tc(z_flat, wt, z_sq, w_sq):
    n, d = z_flat.shape
    k = wt.shape[1]
    grid = (n // _TM, k // _TKC)
    body = functools.partial(_vq_body, n_elem=float(n * d))
    return pl.pallas_call(
        body,
        grid=grid,
        in_specs=[
            pl.BlockSpec((_TM, d), lambda i, j: (i, 0)),
            pl.BlockSpec((d, _TKC), lambda i, j: (0, j)),
            pl.BlockSpec((_TM, 1), lambda i, j: (i, 0)),
            pl.BlockSpec((1, _TKC), lambda i, j: (0, j)),
        ],
        out_specs=[
            pl.BlockSpec((_TM, 1), lambda i, j: (i, 0)),
            pl.BlockSpec((1, 1), lambda i, j: (0, 0)),
            pl.BlockSpec((1, 1), lambda i, j: (0, 0)),
        ],
        out_shape=[
            jax.ShapeDtypeStruct((n, 1), jnp.int32),
            jax.ShapeDtypeStruct((1, 1), jnp.float32),
            jax.ShapeDtypeStruct((1, 1), jnp.float32),
        ],
        scratch_shapes=[
            pltpu.VMEM((_TM, 1), jnp.float32),
            pltpu.VMEM((_TM, 1), jnp.float32),
            pltpu.VMEM((_TM, 1), jnp.int32),
            pltpu.VMEM((1, 1), jnp.float32),
        ],
    )(z_flat, wt, z_sq, w_sq)


def _sc_gather(table, idx):
    v, d0 = table.shape
    d = 128
    table = jnp.pad(table, ((0, 0), (0, d - d0)))
    b = idx.shape[0]
    info = plsc.get_sparse_core_info()
    nc, ns = info.num_cores, info.num_subcores
    nw = nc * ns
    b_per_w = b // nw
    mesh = plsc.VectorSubcoreMesh(core_axis_name="c", subcore_axis_name="s")

    @functools.partial(
        pl.kernel,
        mesh=mesh,
        out_type=jax.ShapeDtypeStruct((b, d), jnp.float32),
        scratch_types=[
            pltpu.VMEM((b_per_w,), jnp.int32),
            pltpu.VMEM((b_per_w, d), jnp.float32),
            pltpu.SemaphoreType.DMA,
        ],
    )
    def k(table_hbm, idx_hbm, out_hbm, idx_v, rows_v, sem):
        wid = lax.axis_index("s") * nc + lax.axis_index("c")
        base = wid * b_per_w
        pltpu.sync_copy(idx_hbm.at[pl.ds(base, b_per_w)], idx_v)
        pltpu.async_copy(table_hbm.at[idx_v], rows_v, sem).wait()
        pltpu.sync_copy(rows_v, out_hbm.at[pl.ds(base, b_per_w)])

    return k(table, idx)[:, :d0]


def kernel(z, W):
    e_dim = z.shape[-1]
    z_flat = z.reshape(-1, e_dim)
    z_sq = jnp.sum(z_flat * z_flat, axis=-1, keepdims=True)
    w_sq = jnp.sum(W * W, axis=-1)[None, :]
    wt = W.T
    idx2, commit, codebook = _vq_tc(-2.0 * z_flat, wt, z_sq, w_sq)
    indices = idx2.reshape(-1)
    z_q = _sc_gather(W, indices)
    z_q_st = z_q.reshape(z.shape)
    return (z_q_st, commit.reshape(()), codebook.reshape(()), indices)


# speedup vs baseline: 1.0389x; 1.0004x over previous
import functools

import jax
import jax.numpy as jnp
from jax import lax
from jax.experimental import pallas as pl
from jax.experimental.pallas import tpu as pltpu
from jax.experimental.pallas import tpu_sc as plsc

_TM = 512
_TKC = 2048
_BETA = 0.25


def _vq_body(zb, wtb, zsqb, wsqb, idx_out, commit_out, codebook_out,
             best_d, best_t, best_i, acc, *, n_elem):
    i = pl.program_id(0)
    j = pl.program_id(1)
    ni = pl.num_programs(0)
    nj = pl.num_programs(1)

    dot = jnp.dot((-2.0 * zb[...]).astype(jnp.bfloat16),
                  wtb[...].astype(jnp.bfloat16),
                  preferred_element_type=jnp.float32)
    d2 = (zsqb[...] + wsqb[...]) + dot
    zdist = jnp.sqrt(jnp.maximum(d2, 0.0))

    bmin = jnp.min(zdist, axis=1, keepdims=True)
    ids = lax.broadcasted_iota(jnp.int32, (_TM, _TKC), 1)
    big = jnp.int32(2**31 - 1)
    bidx = jnp.min(jnp.where(zdist == bmin, ids, big), axis=1,
                   keepdims=True) + j * _TKC
    bmin16 = bmin.astype(jnp.bfloat16).astype(jnp.float32)

    @pl.when(j == 0)
    def _():
        best_d[...] = bmin16
        best_t[...] = bmin
        best_i[...] = bidx

    @pl.when(j > 0)
    def _():
        cur_d = best_d[...]
        cur_i = best_i[...]
        keep = (cur_d < bmin) | ((cur_d == bmin) & (cur_i < bidx))
        best_i[...] = jnp.where(keep, cur_i, bidx)
        best_t[...] = jnp.where(keep, best_t[...], bmin)
        best_d[...] = jnp.where(keep, cur_d, bmin16)

    @pl.when(j == nj - 1)
    def _():
        idx_out[...] = best_i[...]
        bv = best_t[...]
        s = jnp.sum(bv * bv).reshape(1, 1)

        @pl.when(i == 0)
        def _():
            acc[...] = s

        @pl.when(i > 0)
        def _():
            acc[...] = acc[...] + s

        @pl.when(i == ni - 1)
        def _():
            mean_sq = acc[...] / n_elem
            commit_out[...] = _BETA * mean_sq
            codebook_out[...] = mean_sq


def _vq_tc(z_flat, wt, z_sq, w_sq):
    n, d = z_flat.shape
    k = wt.shape[1]
    grid = (n // _TM, k // _TKC)
    body = functools.partial(_vq_body, n_elem=float(n * d))
    return pl.pallas_call(
        body,
        grid=grid,
        in_specs=[
            pl.BlockSpec((_TM, d), lambda i, j: (i, 0)),
            pl.BlockSpec((d, _TKC), lambda i, j: (0, j)),
            pl.BlockSpec((_TM, 1), lambda i, j: (i, 0)),
            pl.BlockSpec((1, _TKC), lambda i, j: (0, j)),
        ],
        out_specs=[
            pl.BlockSpec((_TM, 1), lambda i, j: (i, 0)),
            pl.BlockSpec((1, 1), lambda i, j: (0, 0)),
            pl.BlockSpec((1, 1), lambda i, j: (0, 0)),
        ],
        out_shape=[
            jax.ShapeDtypeStruct((n, 1), jnp.int32),
            jax.ShapeDtypeStruct((1, 1), jnp.float32),
            jax.ShapeDtypeStruct((1, 1), jnp.float32),
        ],
        scratch_shapes=[
            pltpu.VMEM((_TM, 1), jnp.float32),
            pltpu.VMEM((_TM, 1), jnp.float32),
            pltpu.VMEM((_TM, 1), jnp.int32),
            pltpu.VMEM((1, 1), jnp.float32),
        ],
    )(z_flat, wt, z_sq, w_sq)


def _sc_gather(table, idx):
    v, d0 = table.shape
    d = 128
    table = jnp.pad(table, ((0, 0), (0, d - d0)))
    b = idx.shape[0]
    info = plsc.get_sparse_core_info()
    nc, ns = info.num_cores, info.num_subcores
    nw = nc * ns
    b_per_w = b // nw
    mesh = plsc.VectorSubcoreMesh(core_axis_name="c", subcore_axis_name="s")

    @functools.partial(
        pl.kernel,
        mesh=mesh,
        out_type=jax.ShapeDtypeStruct((b, d), jnp.float32),
        scratch_types=[
            pltpu.VMEM((b_per_w,), jnp.int32),
            pltpu.VMEM((b_per_w, d), jnp.float32),
            pltpu.SemaphoreType.DMA,
        ],
    )
    def k(table_hbm, idx_hbm, out_hbm, idx_v, rows_v, sem):
        wid = lax.axis_index("s") * nc + lax.axis_index("c")
        base = wid * b_per_w
        pltpu.sync_copy(idx_hbm.at[pl.ds(base, b_per_w)], idx_v)
        pltpu.async_copy(table_hbm.at[idx_v], rows_v, sem).wait()
        pltpu.sync_copy(rows_v, out_hbm.at[pl.ds(base, b_per_w)])

    return k(table, idx)[:, :d0]


def kernel(z, W):
    e_dim = z.shape[-1]
    z_flat = z.reshape(-1, e_dim)
    z_sq = jnp.sum(z_flat * z_flat, axis=-1, keepdims=True)
    w_sq = jnp.sum(W * W, axis=-1)[None, :]
    wt = W.T
    idx2, commit, codebook = _vq_tc(z_flat, wt, z_sq, w_sq)
    indices = idx2.reshape(-1)
    z_q = _sc_gather(W, indices)
    z_q_st = z_q.reshape(z.shape)
    return (z_q_st, commit.reshape(()), codebook.reshape(()), indices)


# speedup vs baseline: 1.1123x; 1.0707x over previous
import functools

import jax
import jax.numpy as jnp
from jax import lax
from jax.experimental import pallas as pl
from jax.experimental.pallas import tpu as pltpu
from jax.experimental.pallas import tpu_sc as plsc

_TM = 1024
_TKC = 2048
_BETA = 0.25


def _vq_body(zb, wtb, zsqb, wsqb, idx_out, commit_out, codebook_out,
             best_d, best_t, best_i, acc, *, n_elem):
    i = pl.program_id(0)
    j = pl.program_id(1)
    ni = pl.num_programs(0)
    nj = pl.num_programs(1)

    dot = jnp.dot((-2.0 * zb[...]).astype(jnp.bfloat16),
                  wtb[...].astype(jnp.bfloat16),
                  preferred_element_type=jnp.float32)
    d2 = (zsqb[...] + wsqb[...]) + dot
    zdist = jnp.sqrt(jnp.maximum(d2, 0.0))

    bmin = jnp.min(zdist, axis=1, keepdims=True)
    ids = lax.broadcasted_iota(jnp.int32, (_TM, _TKC), 1)
    big = jnp.int32(2**31 - 1)
    bidx = jnp.min(jnp.where(zdist == bmin, ids, big), axis=1,
                   keepdims=True) + j * _TKC
    bmin16 = bmin.astype(jnp.bfloat16).astype(jnp.float32)

    @pl.when(j == 0)
    def _():
        best_d[...] = bmin16
        best_t[...] = bmin
        best_i[...] = bidx

    @pl.when(j > 0)
    def _():
        cur_d = best_d[...]
        cur_i = best_i[...]
        keep = (cur_d < bmin) | ((cur_d == bmin) & (cur_i < bidx))
        best_i[...] = jnp.where(keep, cur_i, bidx)
        best_t[...] = jnp.where(keep, best_t[...], bmin)
        best_d[...] = jnp.where(keep, cur_d, bmin16)

    @pl.when(j == nj - 1)
    def _():
        idx_out[...] = best_i[...]
        bv = best_t[...]
        s = jnp.sum(bv * bv).reshape(1, 1)

        @pl.when(i == 0)
        def _():
            acc[...] = s

        @pl.when(i > 0)
        def _():
            acc[...] = acc[...] + s

        @pl.when(i == ni - 1)
        def _():
            mean_sq = acc[...] / n_elem
            commit_out[...] = _BETA * mean_sq
            codebook_out[...] = mean_sq


def _vq_tc(z_flat, wt, z_sq, w_sq):
    n, d = z_flat.shape
    k = wt.shape[1]
    grid = (n // _TM, k // _TKC)
    body = functools.partial(_vq_body, n_elem=float(n * d))
    return pl.pallas_call(
        body,
        grid=grid,
        in_specs=[
            pl.BlockSpec((_TM, d), lambda i, j: (i, 0)),
            pl.BlockSpec((d, _TKC), lambda i, j: (0, j)),
            pl.BlockSpec((_TM, 1), lambda i, j: (i, 0)),
            pl.BlockSpec((1, _TKC), lambda i, j: (0, j)),
        ],
        out_specs=[
            pl.BlockSpec((_TM, 1), lambda i, j: (i, 0)),
            pl.BlockSpec((1, 1), lambda i, j: (0, 0)),
            pl.BlockSpec((1, 1), lambda i, j: (0, 0)),
        ],
        out_shape=[
            jax.ShapeDtypeStruct((n, 1), jnp.int32),
            jax.ShapeDtypeStruct((1, 1), jnp.float32),
            jax.ShapeDtypeStruct((1, 1), jnp.float32),
        ],
        scratch_shapes=[
            pltpu.VMEM((_TM, 1), jnp.float32),
            pltpu.VMEM((_TM, 1), jnp.float32),
            pltpu.VMEM((_TM, 1), jnp.int32),
            pltpu.VMEM((1, 1), jnp.float32),
        ],
    )(z_flat, wt, z_sq, w_sq)


def _sc_gather(table, idx):
    v, d0 = table.shape
    d = 128
    table = jnp.pad(table, ((0, 0), (0, d - d0)))
    b = idx.shape[0]
    info = plsc.get_sparse_core_info()
    nc, ns = info.num_cores, info.num_subcores
    nw = nc * ns
    b_per_w = b // nw
    mesh = plsc.VectorSubcoreMesh(core_axis_name="c", subcore_axis_name="s")

    @functools.partial(
        pl.kernel,
        mesh=mesh,
        out_type=jax.ShapeDtypeStruct((b, d), jnp.float32),
        scratch_types=[
            pltpu.VMEM((b_per_w,), jnp.int32),
            pltpu.VMEM((b_per_w, d), jnp.float32),
            pltpu.SemaphoreType.DMA,
        ],
    )
    def k(table_hbm, idx_hbm, out_hbm, idx_v, rows_v, sem):
        wid = lax.axis_index("s") * nc + lax.axis_index("c")
        base = wid * b_per_w
        pltpu.sync_copy(idx_hbm.at[pl.ds(base, b_per_w)], idx_v)
        pltpu.async_copy(table_hbm.at[idx_v], rows_v, sem).wait()
        pltpu.sync_copy(rows_v, out_hbm.at[pl.ds(base, b_per_w)])

    return k(table, idx)[:, :d0]


def kernel(z, W):
    e_dim = z.shape[-1]
    z_flat = z.reshape(-1, e_dim)
    z_sq = jnp.sum(z_flat * z_flat, axis=-1, keepdims=True)
    w_sq = jnp.sum(W * W, axis=-1)[None, :]
    wt = W.T
    idx2, commit, codebook = _vq_tc(z_flat, wt, z_sq, w_sq)
    indices = idx2.reshape(-1)
    z_q = _sc_gather(W, indices)
    z_q_st = z_q.reshape(z.shape)
    return (z_q_st, commit.reshape(()), codebook.reshape(()), indices)


# speedup vs baseline: 1.1504x; 1.0342x over previous
import functools

import jax
import jax.numpy as jnp
from jax import lax
from jax.experimental import pallas as pl
from jax.experimental.pallas import tpu as pltpu
from jax.experimental.pallas import tpu_sc as plsc

_TM = 2048
_TKC = 2048
_BETA = 0.25


def _vq_body(zb, wtb, zsqb, wsqb, idx_out, commit_out, codebook_out,
             best_d, best_t, best_i, acc, *, n_elem):
    i = pl.program_id(0)
    j = pl.program_id(1)
    ni = pl.num_programs(0)
    nj = pl.num_programs(1)

    dot = jnp.dot((-2.0 * zb[...]).astype(jnp.bfloat16),
                  wtb[...].astype(jnp.bfloat16),
                  preferred_element_type=jnp.float32)
    d2 = (zsqb[...] + wsqb[...]) + dot
    zdist = jnp.sqrt(jnp.maximum(d2, 0.0))

    bmin = jnp.min(zdist, axis=1, keepdims=True)
    ids = lax.broadcasted_iota(jnp.int32, (_TM, _TKC), 1)
    big = jnp.int32(2**31 - 1)
    bidx = jnp.min(jnp.where(zdist == bmin, ids, big), axis=1,
                   keepdims=True) + j * _TKC
    bmin16 = bmin.astype(jnp.bfloat16).astype(jnp.float32)

    @pl.when(j == 0)
    def _():
        best_d[...] = bmin16
        best_t[...] = bmin
        best_i[...] = bidx

    @pl.when(j > 0)
    def _():
        cur_d = best_d[...]
        cur_i = best_i[...]
        keep = (cur_d < bmin) | ((cur_d == bmin) & (cur_i < bidx))
        best_i[...] = jnp.where(keep, cur_i, bidx)
        best_t[...] = jnp.where(keep, best_t[...], bmin)
        best_d[...] = jnp.where(keep, cur_d, bmin16)

    @pl.when(j == nj - 1)
    def _():
        idx_out[...] = best_i[...]
        bv = best_t[...]
        s = jnp.sum(bv * bv).reshape(1, 1)

        @pl.when(i == 0)
        def _():
            acc[...] = s

        @pl.when(i > 0)
        def _():
            acc[...] = acc[...] + s

        @pl.when(i == ni - 1)
        def _():
            mean_sq = acc[...] / n_elem
            commit_out[...] = _BETA * mean_sq
            codebook_out[...] = mean_sq


def _vq_tc(z_flat, wt, z_sq, w_sq):
    n, d = z_flat.shape
    k = wt.shape[1]
    grid = (n // _TM, k // _TKC)
    body = functools.partial(_vq_body, n_elem=float(n * d))
    return pl.pallas_call(
        body,
        grid=grid,
        in_specs=[
            pl.BlockSpec((_TM, d), lambda i, j: (i, 0)),
            pl.BlockSpec((d, _TKC), lambda i, j: (0, j)),
            pl.BlockSpec((_TM, 1), lambda i, j: (i, 0)),
            pl.BlockSpec((1, _TKC), lambda i, j: (0, j)),
        ],
        out_specs=[
            pl.BlockSpec((_TM, 1), lambda i, j: (i, 0)),
            pl.BlockSpec((1, 1), lambda i, j: (0, 0)),
            pl.BlockSpec((1, 1), lambda i, j: (0, 0)),
        ],
        out_shape=[
            jax.ShapeDtypeStruct((n, 1), jnp.int32),
            jax.ShapeDtypeStruct((1, 1), jnp.float32),
            jax.ShapeDtypeStruct((1, 1), jnp.float32),
        ],
        scratch_shapes=[
            pltpu.VMEM((_TM, 1), jnp.float32),
            pltpu.VMEM((_TM, 1), jnp.float32),
            pltpu.VMEM((_TM, 1), jnp.int32),
            pltpu.VMEM((1, 1), jnp.float32),
        ],
    )(z_flat, wt, z_sq, w_sq)


def _sc_gather(table, idx):
    v, d0 = table.shape
    d = 128
    table = jnp.pad(table, ((0, 0), (0, d - d0)))
    b = idx.shape[0]
    info = plsc.get_sparse_core_info()
    nc, ns = info.num_cores, info.num_subcores
    nw = nc * ns
    b_per_w = b // nw
    mesh = plsc.VectorSubcoreMesh(core_axis_name="c", subcore_axis_name="s")

    @functools.partial(
        pl.kernel,
        mesh=mesh,
        out_type=jax.ShapeDtypeStruct((b, d), jnp.float32),
        scratch_types=[
            pltpu.VMEM((b_per_w,), jnp.int32),
            pltpu.VMEM((b_per_w, d), jnp.float32),
            pltpu.SemaphoreType.DMA,
        ],
    )
    def k(table_hbm, idx_hbm, out_hbm, idx_v, rows_v, sem):
        wid = lax.axis_index("s") * nc + lax.axis_index("c")
        base = wid * b_per_w
        pltpu.sync_copy(idx_hbm.at[pl.ds(base, b_per_w)], idx_v)
        pltpu.async_copy(table_hbm.at[idx_v], rows_v, sem).wait()
        pltpu.sync_copy(rows_v, out_hbm.at[pl.ds(base, b_per_w)])

    return k(table, idx)[:, :d0]


def kernel(z, W):
    e_dim = z.shape[-1]
    z_flat = z.reshape(-1, e_dim)
    z_sq = jnp.sum(z_flat * z_flat, axis=-1, keepdims=True)
    w_sq = jnp.sum(W * W, axis=-1)[None, :]
    wt = W.T
    idx2, commit, codebook = _vq_tc(z_flat, wt, z_sq, w_sq)
    indices = idx2.reshape(-1)
    z_q = _sc_gather(W, indices)
    z_q_st = z_q.reshape(z.shape)
    return (z_q_st, commit.reshape(()), codebook.reshape(()), indices)
